# baseline (speedup 1.0000x reference)
import jax
import jax.numpy as jnp
from jax.experimental import pallas as pl

M = 20000
B = 1024
C, H, W = 3, 32, 32


def _p6_body(i_ref, o_ref):
    o_ref[...] = i_ref[...]


def kernel(buffer_img, buffer_label, x, y, idx, retrieve_idx):
    new_bimg = pl.pallas_call(
        _p6_body,
        grid=(1,),
        in_specs=[pl.BlockSpec((8, C, H, W), lambda i: (0, 0, 0, 0))],
        out_specs=pl.BlockSpec((8, C, H, W), lambda i: (0, 0, 0, 0)),
        out_shape=jax.ShapeDtypeStruct((M, C, H, W), jnp.float32),
        input_output_aliases={0: 0},
    )(buffer_img)
    return (new_bimg, buffer_label, x, y)


# speedup vs baseline: 3.0086x; 3.0086x over previous
import functools

import jax
import jax.numpy as jnp
from jax import lax
from jax.experimental import pallas as pl
from jax.experimental.pallas import tpu as pltpu
from jax.experimental.pallas import tpu_sc as plsc

M = 20000
B = 1024
C, H, W = 3, 32, 32
D = C * H * W

NC, NS = 2, 16
NW = NC * NS
BPW = B // NW
LANES = 16

_mesh = plsc.VectorSubcoreMesh(core_axis_name="c", subcore_axis_name="s")
_sc_params = pltpu.CompilerParams(needs_layout_passes=False)


def _lastmatch_body(a_ref, b_ref, o_ref):
    a = a_ref[...]
    b = b_ref[...]
    eq = a == b
    j = lax.broadcasted_iota(jnp.int32, (B, B), 0)
    o_ref[...] = jnp.max(jnp.where(eq, j, -1), axis=0, keepdims=True)


def _lastmatch(a, b):
    out = pl.pallas_call(
        _lastmatch_body,
        out_shape=jax.ShapeDtypeStruct((1, B), jnp.int32),
    )(a.reshape(B, 1), b.reshape(1, B))
    return out.reshape(B)


def _update_body(buf_ref, lab_ref, xf_ref, y_ref, idx_ref, w_ref, ridx_ref,
                 outlab_ref, rety_ref,
                 idx_v, w_v, rows_v, lab_v, y_v, idxall_v, wall_v, ridx_v,
                 rety_v, sem):
    wid = lax.axis_index("s") * NC + lax.axis_index("c")
    base = wid * BPW

    pltpu.sync_copy(idx_ref.at[pl.ds(base, BPW)], idx_v)
    pltpu.sync_copy(w_ref.at[pl.ds(base, BPW)], w_v)
    pltpu.async_copy(xf_ref.at[w_v], rows_v, sem).wait()
    pltpu.async_copy(rows_v, buf_ref.at[idx_v], sem).wait()

    @pl.when(wid == 0)
    def _():
        pltpu.sync_copy(lab_ref, lab_v)
        pltpu.sync_copy(y_ref, y_v)
        pltpu.sync_copy(idx_ref, idxall_v)
        pltpu.sync_copy(w_ref, wall_v)
        pltpu.sync_copy(ridx_ref, ridx_v)

        def upd(k, carry):
            s = k * LANES
            iv = idxall_v[pl.ds(s, LANES)]
            wv = wall_v[pl.ds(s, LANES)]
            vals = plsc.load_gather(y_v, [wv])
            plsc.store_scatter(lab_v, [iv], vals)
            return carry

        lax.fori_loop(0, B // LANES, upd, 0)

        def ret(k, carry):
            s = k * LANES
            rv = ridx_v[pl.ds(s, LANES)]
            rety_v[pl.ds(s, LANES)] = plsc.load_gather(lab_v, [rv])
            return carry

        lax.fori_loop(0, B // LANES, ret, 0)

        pltpu.sync_copy(lab_v, outlab_ref)
        pltpu.sync_copy(rety_v, rety_ref)


_sc_update = functools.partial(
    pl.kernel,
    out_type=(
        jax.ShapeDtypeStruct((M,), jnp.int32),
        jax.ShapeDtypeStruct((B,), jnp.int32),
    ),
    mesh=_mesh,
    compiler_params=_sc_params,
    scratch_types=[
        pltpu.VMEM((BPW,), jnp.int32),
        pltpu.VMEM((BPW,), jnp.int32),
        pltpu.VMEM((BPW, D), jnp.float32),
        pltpu.VMEM((M,), jnp.int32),
        pltpu.VMEM((B,), jnp.int32),
        pltpu.VMEM((B,), jnp.int32),
        pltpu.VMEM((B,), jnp.int32),
        pltpu.VMEM((B,), jnp.int32),
        pltpu.VMEM((B,), jnp.int32),
        pltpu.SemaphoreType.DMA,
    ],
)(_update_body)


def _gather_body(buf_ref, ridx_ref, out_ref, ridx_v, rows_v, sem):
    wid = lax.axis_index("s") * NC + lax.axis_index("c")
    base = wid * BPW
    pltpu.sync_copy(ridx_ref.at[pl.ds(base, BPW)], ridx_v)
    pltpu.async_copy(buf_ref.at[ridx_v], rows_v, sem).wait()
    pltpu.sync_copy(rows_v, out_ref.at[pl.ds(base, BPW)])


_sc_gather = functools.partial(
    pl.kernel,
    out_type=jax.ShapeDtypeStruct((B, D), jnp.float32),
    mesh=_mesh,
    compiler_params=_sc_params,
    scratch_types=[
        pltpu.VMEM((BPW,), jnp.int32),
        pltpu.VMEM((BPW, D), jnp.float32),
        pltpu.SemaphoreType.DMA,
    ],
)(_gather_body)


def kernel(buffer_img, buffer_label, x, y, idx, retrieve_idx):
    bimg = buffer_img.reshape(M, D)
    xf = x.reshape(B, D)
    w = _lastmatch(idx, idx)

    buf_ref = jax.new_ref(bimg)
    new_lab, ret_y = _sc_update(buf_ref, buffer_label, xf, y, idx, w,
                                retrieve_idx)
    new_bimg = buf_ref[...]
    ret_x = _sc_gather(new_bimg, retrieve_idx)
    return (new_bimg.reshape(M, C, H, W), new_lab,
            ret_x.reshape(B, C, H, W), ret_y)
